# baseline (speedup 1.0000x reference)
import functools

import jax
import jax.numpy as jnp
from jax.experimental import pallas as pl
from jax.experimental.pallas import tpu as pltpu

_REG = 0.01


def _tae_block_kernel(clusts_ref, x_ref, y_ref, c8_ref, we_ref, wd_ref,
                      loss_ref, assign_ref,
                      proxy_buf, mse_buf, en_sum,
                      *, num_blocks, kb, k_total):
    g = pl.program_id(0)

    @pl.when(g == 0)
    def _init():
        en_sum[...] = jnp.zeros(en_sum.shape, jnp.float32)

    x = x_ref[...]
    y = y_ref[...]
    c8 = c8_ref[...]
    we = we_ref[...]
    wd = wd_ref[...]

    b, d = x.shape
    kb_, _, _ = c8.shape
    x4 = x.reshape(1, b // 8, 8, d)
    cb = c8[:, None, :, :]
    xc = (x4 - cb).reshape(kb_, b, d)
    e = jax.lax.dot_general(
        xc, we, (((2,), (1,)), ((0,), (0,))),
        preferred_element_type=jnp.float32)
    o_dot = jax.lax.dot_general(
        e, wd, (((2,), (1,)), ((0,), (0,))),
        preferred_element_type=jnp.float32)
    o = (o_dot.reshape(kb_, b // 8, 8, d) + cb).reshape(kb_, b, d)
    diff = o - y[None, :, :]
    mse = jnp.mean(diff * diff, axis=2)
    en = jnp.sum(e * e, axis=2)

    mse_buf[pl.ds(g * kb, kb), :] = mse
    proxy_buf[pl.ds(g * kb, kb), :] = mse + _REG * en
    en_sum[...] += jnp.sum(en, axis=0, keepdims=True)

    @pl.when(g == num_blocks - 1)
    def _finish():
        pf = proxy_buf[...]
        rows_k = jax.lax.broadcasted_iota(jnp.int32, pf.shape, 0)
        best = jnp.min(pf, axis=0, keepdims=True)
        arg = jnp.min(jnp.where(pf == best, rows_k, k_total),
                      axis=0, keepdims=True)
        assign_ref[...] = (rows_k == arg).astype(jnp.int32)
        hit = rows_k == clusts_ref[...]
        msel = jnp.sum(jnp.where(hit, mse_buf[...], 0.0),
                       axis=0, keepdims=True)
        loss_ref[...] = jnp.sum(msel + _REG * en_sum[...], keepdims=True)


@jax.jit
def kernel(x, y, clusts, centers, W_enc, W_dec):
    B, D = x.shape
    K, _, H = W_enc.shape
    KB = 64
    num_blocks = K // KB
    clusts2 = clusts.astype(jnp.int32).reshape(1, B)

    loss2, assign = pl.pallas_call(
        functools.partial(_tae_block_kernel, num_blocks=num_blocks,
                          kb=KB, k_total=K),
        grid=(num_blocks,),
        in_specs=[
            pl.BlockSpec((1, B), lambda g: (0, 0)),
            pl.BlockSpec((B, D), lambda g: (0, 0)),
            pl.BlockSpec((B, D), lambda g: (0, 0)),
            pl.BlockSpec((KB, 8, D), lambda g: (g, 0, 0)),
            pl.BlockSpec((KB, D, H), lambda g: (g, 0, 0)),
            pl.BlockSpec((KB, H, D), lambda g: (g, 0, 0)),
        ],
        out_specs=[
            pl.BlockSpec((1, 1), lambda g: (0, 0)),
            pl.BlockSpec((K, B), lambda g: (0, 0)),
        ],
        out_shape=[
            jax.ShapeDtypeStruct((1, 1), jnp.float32),
            jax.ShapeDtypeStruct((K, B), jnp.int32),
        ],
        scratch_shapes=[
            pltpu.VMEM((K, B), jnp.float32),
            pltpu.VMEM((K, B), jnp.float32),
            pltpu.VMEM((1, B), jnp.float32),
        ],
        compiler_params=pltpu.CompilerParams(
            dimension_semantics=("arbitrary",),
        ),
    )(clusts2, x, y,
      jnp.broadcast_to(centers[:, None, :], (K, 8, D)), W_enc, W_dec)
    return loss2[0, 0], assign


# speedup vs baseline: 1.0049x; 1.0049x over previous
import functools

import jax
import jax.numpy as jnp
from jax.experimental import pallas as pl
from jax.experimental.pallas import tpu as pltpu

_REG = 0.01


def _tae_block_kernel(clusts_ref, x_ref, y_ref, c_ref, we_ref, wd_ref,
                      loss_ref, assign_ref,
                      proxy_buf, mse_buf, en_sum,
                      *, num_blocks, kb, k_total):
    g = pl.program_id(0)

    @pl.when(g == 0)
    def _init():
        en_sum[...] = jnp.zeros(en_sum.shape, jnp.float32)

    x = x_ref[...]
    y = y_ref[...]
    c = c_ref[...]
    we = we_ref[...]
    wd = wd_ref[...]

    xc = x[None, :, :] - c[:, None, :]
    e = jax.lax.dot_general(
        xc, we, (((2,), (1,)), ((0,), (0,))),
        preferred_element_type=jnp.float32)
    o = jax.lax.dot_general(
        e, wd, (((2,), (1,)), ((0,), (0,))),
        preferred_element_type=jnp.float32) + c[:, None, :]
    diff = o - y[None, :, :]
    mse = jnp.mean(diff * diff, axis=2)
    en = jnp.sum(e * e, axis=2)

    mse_buf[pl.ds(g * kb, kb), :] = mse
    proxy_buf[pl.ds(g * kb, kb), :] = mse + _REG * en
    en_sum[...] += jnp.sum(en, axis=0, keepdims=True)

    @pl.when(g == num_blocks - 1)
    def _finish():
        pf = proxy_buf[...]
        rows_k = jax.lax.broadcasted_iota(jnp.int32, pf.shape, 0)
        best = jnp.min(pf, axis=0, keepdims=True)
        arg = jnp.min(jnp.where(pf == best, rows_k, k_total),
                      axis=0, keepdims=True)
        assign_ref[...] = (rows_k == arg).astype(jnp.int32)
        hit = rows_k == clusts_ref[...]
        msel = jnp.sum(jnp.where(hit, mse_buf[...], 0.0),
                       axis=0, keepdims=True)
        loss_ref[...] = jnp.sum(msel + _REG * en_sum[...], keepdims=True)


@jax.jit
def kernel(x, y, clusts, centers, W_enc, W_dec):
    B, D = x.shape
    K, _, H = W_enc.shape
    KB = 16
    num_blocks = K // KB
    clusts2 = clusts.astype(jnp.int32).reshape(1, B)

    loss2, assign = pl.pallas_call(
        functools.partial(_tae_block_kernel, num_blocks=num_blocks,
                          kb=KB, k_total=K),
        grid=(num_blocks,),
        in_specs=[
            pl.BlockSpec((1, B), lambda g: (0, 0)),
            pl.BlockSpec((B, D), lambda g: (0, 0)),
            pl.BlockSpec((B, D), lambda g: (0, 0)),
            pl.BlockSpec((KB, D), lambda g: (g, 0)),
            pl.BlockSpec((KB, D, H), lambda g: (g, 0, 0)),
            pl.BlockSpec((KB, H, D), lambda g: (g, 0, 0)),
        ],
        out_specs=[
            pl.BlockSpec((1, 1), lambda g: (0, 0)),
            pl.BlockSpec((K, B), lambda g: (0, 0)),
        ],
        out_shape=[
            jax.ShapeDtypeStruct((1, 1), jnp.float32),
            jax.ShapeDtypeStruct((K, B), jnp.int32),
        ],
        scratch_shapes=[
            pltpu.VMEM((K, B), jnp.float32),
            pltpu.VMEM((K, B), jnp.float32),
            pltpu.VMEM((1, B), jnp.float32),
        ],
        compiler_params=pltpu.CompilerParams(
            dimension_semantics=("arbitrary",),
        ),
    )(clusts2, x, y, centers, W_enc, W_dec)
    return loss2[0, 0], assign


# speedup vs baseline: 1.0343x; 1.0293x over previous
import functools

import jax
import jax.numpy as jnp
from jax.experimental import pallas as pl
from jax.experimental.pallas import tpu as pltpu

_REG = 0.01


def _tae_block_kernel(clusts_ref, x_ref, y_ref, c_ref, we_ref, wd_ref,
                      loss_ref, assign_ref,
                      proxy_buf, mse_buf, en_sum,
                      *, num_blocks, kb, k_total):
    g = pl.program_id(0)

    @pl.when(g == 0)
    def _init():
        en_sum[...] = jnp.zeros(en_sum.shape, jnp.float32)

    x = x_ref[...]
    y = y_ref[...]
    c = c_ref[...]
    we = we_ref[...]
    wd = wd_ref[...]

    xc = x[None, :, :] - c[:, None, :]
    e = jax.lax.dot_general(
        xc, we, (((2,), (1,)), ((0,), (0,))),
        preferred_element_type=jnp.float32)
    o = jax.lax.dot_general(
        e, wd, (((2,), (1,)), ((0,), (0,))),
        preferred_element_type=jnp.float32) + c[:, None, :]
    diff = o - y[None, :, :]
    mse = jnp.mean(diff * diff, axis=2)
    en = jnp.sum(e * e, axis=2)

    mse_buf[g] = mse
    proxy_buf[g] = mse + _REG * en
    en_sum[...] += jnp.sum(en, axis=0, keepdims=True)

    @pl.when(g == num_blocks - 1)
    def _finish():
        pf = proxy_buf[...].reshape(k_total, -1)
        rows_k = jax.lax.broadcasted_iota(jnp.int32, pf.shape, 0)
        best = jnp.min(pf, axis=0, keepdims=True)
        arg = jnp.min(jnp.where(pf == best, rows_k, k_total),
                      axis=0, keepdims=True)
        assign_ref[...] = (rows_k == arg).astype(jnp.int32)
        hit = rows_k == clusts_ref[...]
        msel = jnp.sum(jnp.where(hit, mse_buf[...].reshape(k_total, -1), 0.0),
                       axis=0, keepdims=True)
        loss_ref[...] = jnp.sum(msel + _REG * en_sum[...], keepdims=True)


@jax.jit
def kernel(x, y, clusts, centers, W_enc, W_dec):
    B, D = x.shape
    K, _, H = W_enc.shape
    KB = 64
    num_blocks = K // KB
    clusts2 = clusts.astype(jnp.int32).reshape(1, B)

    loss2, assign = pl.pallas_call(
        functools.partial(_tae_block_kernel, num_blocks=num_blocks,
                          kb=KB, k_total=K),
        grid=(num_blocks,),
        in_specs=[
            pl.BlockSpec((1, B), lambda g: (0, 0)),
            pl.BlockSpec((B, D), lambda g: (0, 0)),
            pl.BlockSpec((B, D), lambda g: (0, 0)),
            pl.BlockSpec((KB, D), lambda g: (g, 0)),
            pl.BlockSpec((KB, D, H), lambda g: (g, 0, 0)),
            pl.BlockSpec((KB, H, D), lambda g: (g, 0, 0)),
        ],
        out_specs=[
            pl.BlockSpec((1, 1), lambda g: (0, 0)),
            pl.BlockSpec((K, B), lambda g: (0, 0)),
        ],
        out_shape=[
            jax.ShapeDtypeStruct((1, 1), jnp.float32),
            jax.ShapeDtypeStruct((K, B), jnp.int32),
        ],
        scratch_shapes=[
            pltpu.VMEM((num_blocks, KB, B), jnp.float32),
            pltpu.VMEM((num_blocks, KB, B), jnp.float32),
            pltpu.VMEM((1, B), jnp.float32),
        ],
        compiler_params=pltpu.CompilerParams(
            dimension_semantics=("arbitrary",),
        ),
    )(clusts2, x, y, centers, W_enc, W_dec)
    return loss2[0, 0], assign
